# baseline (speedup 1.0000x reference)
import functools

import jax
import jax.numpy as jnp
from jax.experimental import pallas as pl


def _xw_kernel(x_ref, w_ref, out_ref):
    out_ref[...] = jnp.dot(x_ref[...], w_ref[...],
                           preferred_element_type=jnp.float32)


def _layer_kernel(a_ref, h_ref, b_ref, w2_ref, out_ref, *, project):
    s = jnp.dot(a_ref[...], h_ref[...], preferred_element_type=jnp.float32)
    s = jnp.maximum(s + b_ref[...], 0.0)
    if project:
        s = jnp.dot(s, w2_ref[...], preferred_element_type=jnp.float32)
    out_ref[...] = s


def _layer(a, h, b, w2, *, project, bm):
    n = a.shape[0]
    f = h.shape[1]
    grid = (n // bm,)
    in_specs = [
        pl.BlockSpec((bm, n), lambda i: (i, 0)),
        pl.BlockSpec((n, f), lambda i: (0, 0)),
        pl.BlockSpec((1, f), lambda i: (0, 0)),
        pl.BlockSpec(w2.shape, lambda i: (0, 0)),
    ]
    return pl.pallas_call(
        functools.partial(_layer_kernel, project=project),
        grid=grid,
        in_specs=in_specs,
        out_specs=pl.BlockSpec((bm, f), lambda i: (i, 0)),
        out_shape=jax.ShapeDtypeStruct((n, f), jnp.float32),
    )(a, h, b.reshape(1, f), w2)


def kernel(x, a, W1, b1, W2, b2):
    h0 = pl.pallas_call(
        _xw_kernel,
        out_shape=jax.ShapeDtypeStruct((x.shape[0], W1.shape[1]), jnp.float32),
    )(x, W1)
    g = _layer(a, h0, b1, W2, project=True, bm=80)
    out = _layer(a, g, b2, W2, project=False, bm=80)
    return out


# speedup vs baseline: 1.3674x; 1.3674x over previous
import functools

import jax
import jax.numpy as jnp
from jax.experimental import pallas as pl


def _xw_kernel(x_ref, w_ref, out_ref):
    out_ref[...] = jnp.dot(x_ref[...], w_ref[...],
                           preferred_element_type=jnp.float32)


def _layer_kernel(a_ref, h_ref, b_ref, w2_ref, out_ref, *, project):
    s = jnp.dot(a_ref[...], h_ref[...], preferred_element_type=jnp.float32)
    s = jnp.maximum(s + b_ref[...], 0.0)
    if project:
        s = jnp.dot(s, w2_ref[...], preferred_element_type=jnp.float32)
    out_ref[...] = s


def _layer(a, h, b, w2, *, project, bm):
    n = a.shape[0]
    f = h.shape[1]
    grid = (n // bm,)
    in_specs = [
        pl.BlockSpec((bm, n), lambda i: (i, 0)),
        pl.BlockSpec((n, f), lambda i: (0, 0)),
        pl.BlockSpec((1, f), lambda i: (0, 0)),
        pl.BlockSpec(w2.shape, lambda i: (0, 0)),
    ]
    return pl.pallas_call(
        functools.partial(_layer_kernel, project=project),
        grid=grid,
        in_specs=in_specs,
        out_specs=pl.BlockSpec((bm, f), lambda i: (i, 0)),
        out_shape=jax.ShapeDtypeStruct((n, f), jnp.float32),
    )(a, h, b.reshape(1, f), w2)


def kernel(x, a, W1, b1, W2, b2):
    h0 = pl.pallas_call(
        _xw_kernel,
        out_shape=jax.ShapeDtypeStruct((x.shape[0], W1.shape[1]), jnp.float32),
    )(x, W1)
    g = _layer(a, h0, b1, W2, project=True, bm=400)
    out = _layer(a, g, b2, W2, project=False, bm=400)
    return out


# speedup vs baseline: 1.4433x; 1.0555x over previous
import functools

import jax
import jax.numpy as jnp
from jax.experimental import pallas as pl
from jax.experimental.pallas import tpu as pltpu


def _gcn_kernel(x_ref, a_ref, w1_ref, b1_ref, w2_ref, b2_ref, out_ref,
                h0_ref, g_ref, *, bm, nm):
    i = pl.program_id(0)
    m = jax.lax.rem(i, nm)

    @pl.when(i == 0)
    def _():
        h0_ref[...] = jnp.dot(x_ref[...], w1_ref[...],
                              preferred_element_type=jnp.float32)

    @pl.when(i < nm)
    def _():
        s = jnp.dot(a_ref[...], h0_ref[...],
                    preferred_element_type=jnp.float32)
        s = jnp.maximum(s + b1_ref[...], 0.0)
        g_ref[pl.ds(m * bm, bm), :] = jnp.dot(
            s, w2_ref[...], preferred_element_type=jnp.float32)

    @pl.when(i >= nm)
    def _():
        s = jnp.dot(a_ref[...], g_ref[...],
                    preferred_element_type=jnp.float32)
        out_ref[...] = jnp.maximum(s + b2_ref[...], 0.0)


def kernel(x, a, W1, b1, W2, b2):
    n, f_in = x.shape
    f = W1.shape[1]
    bm = 400
    nm = n // bm
    return pl.pallas_call(
        functools.partial(_gcn_kernel, bm=bm, nm=nm),
        grid=(2 * nm,),
        in_specs=[
            pl.BlockSpec((n, f_in), lambda i: (0, 0)),
            pl.BlockSpec((bm, n), lambda i, nm=nm: (i % nm, 0)),
            pl.BlockSpec((f_in, f), lambda i: (0, 0)),
            pl.BlockSpec((1, f), lambda i: (0, 0)),
            pl.BlockSpec((f, f), lambda i: (0, 0)),
            pl.BlockSpec((1, f), lambda i: (0, 0)),
        ],
        out_specs=pl.BlockSpec(
            (bm, f), lambda i, nm=nm: (jnp.maximum(i - nm, 0), 0)),
        out_shape=jax.ShapeDtypeStruct((n, f), jnp.float32),
        scratch_shapes=[
            pltpu.VMEM((n, f), jnp.float32),
            pltpu.VMEM((n, f), jnp.float32),
        ],
        compiler_params=pltpu.CompilerParams(
            dimension_semantics=("arbitrary",)),
    )(x, a, W1, b1.reshape(1, f), W2, b2.reshape(1, f))
